# baseline (speedup 1.0000x reference)
import functools

import jax
import jax.numpy as jnp
from jax import lax
from jax.experimental import pallas as pl
from jax.experimental.pallas import tpu as pltpu
from jax.experimental.pallas import tpu_sc as plsc

_B = 2
_C = 64
_D = 64
_N = 100000
_S = _B * _D * _D * _D
_NW = 32
_SLOTS_W = _S // _NW
_SENT = _N
_NPAD = _N + 8
_CH = 2000
_NCHUNK = _N // _CH
_GRP = _CH // 16
_CSLOTS = 128
_NCC = _SLOTS_W // _CSLOTS
_SPB = _D * _D * _D


def _sc_kernel(feat, bx, xs, ys, zs, out, m2, ib, ix, iy, iz, g0, g1, obuf,
               sem0, sem1):
    wid = lax.axis_index("s") * 2 + lax.axis_index("c")
    base = wid * _SLOTS_W
    b_of = base // _SPB
    s_of = base % _SPB

    iota = lax.iota(jnp.int32, 16)
    sent = jnp.full((16,), _SENT, dtype=jnp.int32)

    def init_row(r, _):
        for g in range(8):
            m2[r, pl.ds(g * 16, 16)] = sent
        return 0

    lax.fori_loop(0, 128, init_row, 0)

    def scan_chunk(t, _):
        off = t * _CH
        pltpu.sync_copy(bx.at[pl.ds(off, _CH)], ib)
        pltpu.sync_copy(xs.at[pl.ds(off, _CH)], ix)
        pltpu.sync_copy(ys.at[pl.ds(off, _CH)], iy)
        pltpu.sync_copy(zs.at[pl.ds(off, _CH)], iz)

        def grp(g, _):
            b = ib[pl.ds(g * 16, 16)]
            x = ix[pl.ds(g * 16, 16)]
            y = iy[pl.ds(g * 16, 16)]
            z = iz[pl.ds(g * 16, 16)]
            slot = (((b * _D + x) * _D + y) * _D) + z
            loc = slot - base
            ok = (loc >= 0) & (loc < _SLOTS_W)
            locc = loc & (_SLOTS_W - 1)
            row = locc >> 7
            col = locc & 127
            pidx = iota + (off + g * 16)
            plsc.store_scatter(m2, [row, col], pidx, mask=ok)
            return 0

        lax.fori_loop(0, _GRP, grp, 0)
        return 0

    lax.fori_loop(0, _NCHUNK, scan_chunk, 0)

    def fire(k, gbuf, sem):
        pltpu.async_copy(feat.at[m2.at[k]], gbuf, sem)

    def drain(gbuf, sem):
        pltpu.make_async_copy(feat.at[m2.at[0]], gbuf, sem).wait()

    def emit(k, gbuf):
        def t_row(j, _):
            colj = jnp.full((16,), 0, dtype=jnp.int32) + j
            for q in range(4):
                v = gbuf[j, pl.ds(q * 16, 16)]
                plsc.store_scatter(obuf, [iota + q * 16, colj], v)
            return 0

        lax.fori_loop(0, _CSLOTS, t_row, 0)
        pltpu.sync_copy(obuf, out.at[b_of, :, pl.ds(s_of + k * _CSLOTS,
                                                    _CSLOTS)])

    fire(0, g0, sem0)
    fire(1, g1, sem1)

    def chunk_pair(kk, _):
        k0 = kk * 2
        drain(g0, sem0)
        emit(k0, g0)

        @pl.when(kk < (_NCC // 2) - 1)
        def _():
            fire(k0 + 2, g0, sem0)

        drain(g1, sem1)
        emit(k0 + 1, g1)

        @pl.when(kk < (_NCC // 2) - 1)
        def _():
            fire(k0 + 3, g1, sem1)

        return 0

    lax.fori_loop(0, _NCC // 2, chunk_pair, 0)


@functools.cache
def _build():
    @functools.partial(
        pl.kernel,
        out_type=jax.ShapeDtypeStruct((_B, _C, _SPB), jnp.float32),
        mesh=plsc.VectorSubcoreMesh(core_axis_name="c", subcore_axis_name="s"),
        compiler_params=pltpu.CompilerParams(needs_layout_passes=False,
                                             use_tc_tiling_on_sc=False),
        scratch_types=[
            pltpu.VMEM((128, 128), jnp.int32),
            pltpu.VMEM((_CH,), jnp.int32),
            pltpu.VMEM((_CH,), jnp.int32),
            pltpu.VMEM((_CH,), jnp.int32),
            pltpu.VMEM((_CH,), jnp.int32),
            pltpu.VMEM((_CSLOTS, _C), jnp.float32),
            pltpu.VMEM((_CSLOTS, _C), jnp.float32),
            pltpu.VMEM((_C, _CSLOTS), jnp.float32),
            pltpu.SemaphoreType.DMA,
            pltpu.SemaphoreType.DMA,
        ],
    )
    def _sparse_to_dense(feat, bx, xs, ys, zs, out, *scratch):
        _sc_kernel(feat, bx, xs, ys, zs, out, *scratch)

    return _sparse_to_dense


def kernel(features, batch_idx, coords):
    _sparse_to_dense = _build()
    featpad = jnp.concatenate(
        [features, jnp.zeros((_NPAD - _N, _C), dtype=features.dtype)], axis=0)
    dense = _sparse_to_dense(featpad, batch_idx, coords[:, 0], coords[:, 1],
                             coords[:, 2])
    return dense.reshape(_B, _C, _D, _D, _D)


# speedup vs baseline: 1.0001x; 1.0001x over previous
import functools

import jax
import jax.numpy as jnp
from jax import lax
from jax.experimental import pallas as pl
from jax.experimental.pallas import tpu as pltpu
from jax.experimental.pallas import tpu_sc as plsc

_B = 2
_C = 64
_D = 64
_N = 100000
_S = _B * _D * _D * _D
_NW = 32
_SLOTS_W = _S // _NW
_SENT = _N
_NPAD = _N + 8
_CH = 2000
_NCHUNK = _N // _CH
_GRP = _CH // 16
_CSLOTS = 128
_NCC = _SLOTS_W // _CSLOTS
_SPB = _D * _D * _D


def _sc_kernel(feat, bx, xs, ys, zs, out, m2, ib, ix, iy, iz, g0, g1, obuf,
               sem0, sem1):
    wid = lax.axis_index("s") * 2 + lax.axis_index("c")
    base = wid * _SLOTS_W
    b_of = base // _SPB
    s_of = base % _SPB

    iota = lax.iota(jnp.int32, 16)
    sent = jnp.full((16,), _SENT, dtype=jnp.int32)

    def init_row(r, _):
        for g in range(8):
            m2[r, pl.ds(g * 16, 16)] = sent
        return 0

    lax.fori_loop(0, 128, init_row, 0)

    def scan_chunk(t, _):
        off = t * _CH
        pltpu.sync_copy(bx.at[pl.ds(off, _CH)], ib)
        pltpu.sync_copy(xs.at[pl.ds(off, _CH)], ix)
        pltpu.sync_copy(ys.at[pl.ds(off, _CH)], iy)
        pltpu.sync_copy(zs.at[pl.ds(off, _CH)], iz)

        def grp(g, _):
            b = ib[pl.ds(g * 16, 16)]
            x = ix[pl.ds(g * 16, 16)]
            y = iy[pl.ds(g * 16, 16)]
            z = iz[pl.ds(g * 16, 16)]
            slot = (((b * _D + x) * _D + y) * _D) + z
            loc = slot - base
            ok = (loc >= 0) & (loc < _SLOTS_W)
            locc = loc & (_SLOTS_W - 1)
            row = locc >> 7
            col = locc & 127
            pidx = iota + (off + g * 16)
            plsc.store_scatter(m2, [row, col], pidx, mask=ok)
            return 0

        lax.fori_loop(0, _GRP, grp, 0)
        return 0

    with jax.named_scope("scan"):
        lax.fori_loop(0, _NCHUNK, scan_chunk, 0)

    def fire(k, gbuf, sem):
        pltpu.async_copy(feat.at[m2.at[k]], gbuf, sem)

    def drain(gbuf, sem):
        pltpu.make_async_copy(feat.at[m2.at[0]], gbuf, sem).wait()

    def emit(k, gbuf):
        def t_row(j, _):
            colj = jnp.full((16,), 0, dtype=jnp.int32) + j
            for q in range(4):
                v = gbuf[j, pl.ds(q * 16, 16)]
                plsc.store_scatter(obuf, [iota + q * 16, colj], v)
            return 0

        with jax.named_scope("tpose"):
            lax.fori_loop(0, _CSLOTS, t_row, 0)
        with jax.named_scope("out_dma"):
            pltpu.sync_copy(obuf, out.at[b_of, :, pl.ds(s_of + k * _CSLOTS,
                                                        _CSLOTS)])

    fire(0, g0, sem0)
    fire(1, g1, sem1)

    def chunk_pair(kk, _):
        k0 = kk * 2
        with jax.named_scope("g_wait"):
            drain(g0, sem0)
        emit(k0, g0)

        @pl.when(kk < (_NCC // 2) - 1)
        def _():
            fire(k0 + 2, g0, sem0)

        with jax.named_scope("g_wait"):
            drain(g1, sem1)
        emit(k0 + 1, g1)

        @pl.when(kk < (_NCC // 2) - 1)
        def _():
            fire(k0 + 3, g1, sem1)

        return 0

    lax.fori_loop(0, _NCC // 2, chunk_pair, 0)


@functools.cache
def _build():
    @functools.partial(
        pl.kernel,
        out_type=jax.ShapeDtypeStruct((_B, _C, _SPB), jnp.float32),
        mesh=plsc.VectorSubcoreMesh(core_axis_name="c", subcore_axis_name="s"),
        compiler_params=pltpu.CompilerParams(needs_layout_passes=False,
                                             use_tc_tiling_on_sc=False),
        scratch_types=[
            pltpu.VMEM((128, 128), jnp.int32),
            pltpu.VMEM((_CH,), jnp.int32),
            pltpu.VMEM((_CH,), jnp.int32),
            pltpu.VMEM((_CH,), jnp.int32),
            pltpu.VMEM((_CH,), jnp.int32),
            pltpu.VMEM((_CSLOTS, _C), jnp.float32),
            pltpu.VMEM((_CSLOTS, _C), jnp.float32),
            pltpu.VMEM((_C, _CSLOTS), jnp.float32),
            pltpu.SemaphoreType.DMA,
            pltpu.SemaphoreType.DMA,
        ],
    )
    def _sparse_to_dense(feat, bx, xs, ys, zs, out, *scratch):
        _sc_kernel(feat, bx, xs, ys, zs, out, *scratch)

    return _sparse_to_dense


def kernel(features, batch_idx, coords):
    _sparse_to_dense = _build()
    featpad = jnp.concatenate(
        [features, jnp.zeros((_NPAD - _N, _C), dtype=features.dtype)], axis=0)
    dense = _sparse_to_dense(featpad, batch_idx, coords[:, 0], coords[:, 1],
                             coords[:, 2])
    return dense.reshape(_B, _C, _D, _D, _D)


# speedup vs baseline: 11.5423x; 11.5412x over previous
import functools

import jax
import jax.numpy as jnp
from jax import lax
from jax.experimental import pallas as pl
from jax.experimental.pallas import tpu as pltpu
from jax.experimental.pallas import tpu_sc as plsc

_B = 2
_C = 64
_D = 64
_N = 100000
_S = _B * _D * _D * _D
_SPB = _D * _D * _D
_NT = 16
_SLOTS_T = _SPB // _NT
_HALF = _SLOTS_T // 2
_NPAD = 100352
_SENT = _N
_CH = 2000
_NCHUNK = _N // _CH
_GRP = _CH // 16
_CS = 4096
_CPP = _SPB // _CS
_PLANES = 4
_NCK = _PLANES * _CPP
_TB = 1024


def _tc_transpose_body(x_ref, o_ref):
    i = pl.program_id(0)
    rows = jax.lax.broadcasted_iota(jnp.int32, (_TB, _C), 0) + i * _TB
    x = jnp.where(rows < _N, x_ref[...], 0.0)
    o_ref[...] = x.T


def _transpose_features(features):
    return pl.pallas_call(
        _tc_transpose_body,
        grid=(_NPAD // _TB,),
        in_specs=[pl.BlockSpec((_TB, _C), lambda i: (i, 0))],
        out_specs=pl.BlockSpec((_C, _TB), lambda i: (0, i)),
        out_shape=jax.ShapeDtypeStruct((_C, _NPAD), jnp.float32),
    )(features)


def _sc_body(dest, featT, out, m_out, mloc, db0, db1, mb0, mb1, ob0, ob1,
             row, semd0, semd1, semm0, semm1, semo0, semo1):
    sc = lax.axis_index("c")
    tid = lax.axis_index("s")
    mybase = sc * _SPB + tid * _SLOTS_T

    iota = lax.iota(jnp.int32, 16)
    sent = jnp.full((16,), _SENT, dtype=jnp.int32)

    def build_pass(p):
        lo = mybase + p * _HALF

        def init(i, _):
            for u in range(8):
                mloc[pl.ds((i * 8 + u) * 16, 16)] = sent
            return 0

        lax.fori_loop(0, _HALF // 128, init, 0)

        def consume(t, dbuf):
            off = t * _CH

            def grp(g, _):
                d = dbuf[pl.ds(g * 16, 16)]
                loc = d - lo
                ok = (loc >= 0) & (loc < _HALF)
                locc = loc & (_HALF - 1)
                pidx = iota + (off + g * 16)
                plsc.store_scatter(mloc, [locc], pidx, mask=ok)
                return 0

            lax.fori_loop(0, _GRP, grp, 0)

        pltpu.async_copy(dest.at[pl.ds(0, _CH)], db0, semd0)
        pltpu.async_copy(dest.at[pl.ds(_CH, _CH)], db1, semd1)

        def pair(kk, _):
            t0 = kk * 2
            pltpu.make_async_copy(dest.at[pl.ds(0, _CH)], db0, semd0).wait()
            consume(t0, db0)

            @pl.when(t0 + 2 < _NCHUNK)
            def _():
                pltpu.async_copy(dest.at[pl.ds((t0 + 2) * _CH, _CH)], db0,
                                 semd0)

            pltpu.make_async_copy(dest.at[pl.ds(0, _CH)], db1, semd1).wait()
            consume(t0 + 1, db1)

            @pl.when(t0 + 3 < _NCHUNK)
            def _():
                pltpu.async_copy(dest.at[pl.ds((t0 + 3) * _CH, _CH)], db1,
                                 semd1)

            return 0

        lax.fori_loop(0, _NCHUNK // 2, pair, 0)
        pltpu.sync_copy(mloc, m_out.at[pl.ds(lo, _HALF)])

    build_pass(0)
    build_pass(1)
    plsc.subcore_barrier()

    def m_src(k):
        q = k & (_CPP - 1)
        return m_out.at[pl.ds(sc * _SPB + q * _CS, _CS)]

    def emit(k, mbuf, obuf, semo):
        @pl.when((k & (_CPP - 1)) == 0)
        def _():
            c = tid * _PLANES + (k >> 6)
            pltpu.sync_copy(featT.at[c], row)

        def gather(i, _):
            for u in range(8):
                g = i * 8 + u
                idx = mbuf[pl.ds(g * 16, 16)]
                obuf[pl.ds(g * 16, 16)] = plsc.load_gather(row, [idx])
            return 0

        lax.fori_loop(0, _CS // 128, gather, 0)
        c = tid * _PLANES + (k >> 6)
        q = k & (_CPP - 1)
        pltpu.async_copy(obuf, out.at[sc, c, pl.ds(q * _CS, _CS)], semo)

    def o_drain(obuf, semo):
        pltpu.make_async_copy(obuf, out.at[sc, 0, pl.ds(0, _CS)],
                              semo).wait()

    pltpu.async_copy(m_src(0), mb0, semm0)
    pltpu.async_copy(m_src(1), mb1, semm1)

    def chunk_pair(kk, _):
        k0 = kk * 2
        pltpu.make_async_copy(m_src(0), mb0, semm0).wait()

        @pl.when(kk > 0)
        def _():
            o_drain(ob0, semo0)

        emit(k0, mb0, ob0, semo0)

        @pl.when(k0 + 2 < _NCK)
        def _():
            pltpu.async_copy(m_src(k0 + 2), mb0, semm0)

        pltpu.make_async_copy(m_src(0), mb1, semm1).wait()

        @pl.when(kk > 0)
        def _():
            o_drain(ob1, semo1)

        emit(k0 + 1, mb1, ob1, semo1)

        @pl.when(k0 + 3 < _NCK)
        def _():
            pltpu.async_copy(m_src(k0 + 3), mb1, semm1)

        return 0

    lax.fori_loop(0, _NCK // 2, chunk_pair, 0)
    o_drain(ob0, semo0)
    o_drain(ob1, semo1)


@functools.cache
def _build_sc():
    @functools.partial(
        pl.kernel,
        out_type=(
            jax.ShapeDtypeStruct((_B, _C, _SPB), jnp.float32),
            jax.ShapeDtypeStruct((_S,), jnp.int32),
        ),
        mesh=plsc.VectorSubcoreMesh(core_axis_name="c", subcore_axis_name="s"),
        compiler_params=pltpu.CompilerParams(needs_layout_passes=False,
                                             use_tc_tiling_on_sc=False),
        scratch_types=[
            pltpu.VMEM((_HALF,), jnp.int32),
            pltpu.VMEM((_CH,), jnp.int32),
            pltpu.VMEM((_CH,), jnp.int32),
            pltpu.VMEM((_CS,), jnp.int32),
            pltpu.VMEM((_CS,), jnp.int32),
            pltpu.VMEM((_CS,), jnp.float32),
            pltpu.VMEM((_CS,), jnp.float32),
            pltpu.VMEM((_NPAD,), jnp.float32),
            pltpu.SemaphoreType.DMA,
            pltpu.SemaphoreType.DMA,
            pltpu.SemaphoreType.DMA,
            pltpu.SemaphoreType.DMA,
            pltpu.SemaphoreType.DMA,
            pltpu.SemaphoreType.DMA,
        ],
    )
    def _sparse_to_dense(dest, featT, out, m_out, *scratch):
        _sc_body(dest, featT, out, m_out, *scratch)

    return _sparse_to_dense


def kernel(features, batch_idx, coords):
    dest = ((batch_idx * _D + coords[:, 0]) * _D + coords[:, 1]) * _D \
        + coords[:, 2]
    featT = _transpose_features(features)
    dense, _ = _build_sc()(dest.astype(jnp.int32), featT)
    return dense.reshape(_B, _C, _D, _D, _D)


# speedup vs baseline: 11.5809x; 1.0033x over previous
import functools

import jax
import jax.numpy as jnp
from jax import lax
from jax.experimental import pallas as pl
from jax.experimental.pallas import tpu as pltpu
from jax.experimental.pallas import tpu_sc as plsc

_B = 2
_C = 64
_D = 64
_N = 100000
_S = _B * _D * _D * _D
_SPB = _D * _D * _D
_NT = 16
_SLOTS_T = _SPB // _NT
_HALF = _SLOTS_T // 2
_NPAD = 100352
_SENT = _N
_CH = 2000
_NCHUNK = _N // _CH
_GRP = _CH // 16
_CS = 4096
_CPP = _SPB // _CS
_PLANES = 4
_NCK = _PLANES * _CPP
_TB = 1024


def _tc_transpose_body(x_ref, o_ref):
    i = pl.program_id(0)
    rows = jax.lax.broadcasted_iota(jnp.int32, (_TB, _C), 0) + i * _TB
    x = jnp.where(rows < _N, x_ref[...], 0.0)
    o_ref[...] = x.T


def _transpose_features(features):
    return pl.pallas_call(
        _tc_transpose_body,
        grid=(_NPAD // _TB,),
        in_specs=[pl.BlockSpec((_TB, _C), lambda i: (i, 0))],
        out_specs=pl.BlockSpec((_C, _TB), lambda i: (0, i)),
        out_shape=jax.ShapeDtypeStruct((_C, _NPAD), jnp.float32),
    )(features)


def _sc_body(dest, featT, out, m_out, mloc, db0, db1, mb0, mb1, ob0, ob1,
             row, semd0, semd1, semm0, semm1, semo0, semo1):
    sc = lax.axis_index("c")
    tid = lax.axis_index("s")
    mybase = sc * _SPB + tid * _SLOTS_T

    iota = lax.iota(jnp.int32, 16)
    sent = jnp.full((16,), _SENT, dtype=jnp.int32)

    def build_pass(p):
        lo = mybase + p * _HALF

        def init(i, _):
            for u in range(8):
                mloc[pl.ds((i * 8 + u) * 16, 16)] = sent
            return 0

        lax.fori_loop(0, _HALF // 128, init, 0)

        def consume(t, dbuf):
            off = t * _CH

            def grp(g, _):
                d = dbuf[pl.ds(g * 16, 16)]
                loc = d - lo
                ok = (loc >= 0) & (loc < _HALF)
                locc = loc & (_HALF - 1)
                pidx = iota + (off + g * 16)
                plsc.store_scatter(mloc, [locc], pidx, mask=ok)
                return 0

            lax.fori_loop(0, _GRP, grp, 0)

        pltpu.async_copy(dest.at[pl.ds(0, _CH)], db0, semd0)
        pltpu.async_copy(dest.at[pl.ds(_CH, _CH)], db1, semd1)

        def pair(kk, _):
            t0 = kk * 2
            pltpu.make_async_copy(dest.at[pl.ds(0, _CH)], db0, semd0).wait()
            consume(t0, db0)

            @pl.when(t0 + 2 < _NCHUNK)
            def _():
                pltpu.async_copy(dest.at[pl.ds((t0 + 2) * _CH, _CH)], db0,
                                 semd0)

            pltpu.make_async_copy(dest.at[pl.ds(0, _CH)], db1, semd1).wait()
            consume(t0 + 1, db1)

            @pl.when(t0 + 3 < _NCHUNK)
            def _():
                pltpu.async_copy(dest.at[pl.ds((t0 + 3) * _CH, _CH)], db1,
                                 semd1)

            return 0

        lax.fori_loop(0, _NCHUNK // 2, pair, 0)
        pltpu.sync_copy(mloc, m_out.at[pl.ds(lo, _HALF)])

    with jax.named_scope("build"):
        build_pass(0)
        build_pass(1)
        plsc.subcore_barrier()

    def m_src(k):
        q = k & (_CPP - 1)
        return m_out.at[pl.ds(sc * _SPB + q * _CS, _CS)]

    def emit(k, mbuf, obuf, semo):
        @pl.when((k & (_CPP - 1)) == 0)
        def _():
            with jax.named_scope("row"):
                c = tid * _PLANES + (k >> 6)
                pltpu.sync_copy(featT.at[c], row)

        def gather(i, _):
            for u in range(8):
                g = i * 8 + u
                idx = mbuf[pl.ds(g * 16, 16)]
                obuf[pl.ds(g * 16, 16)] = plsc.load_gather(row, [idx])
            return 0

        with jax.named_scope("gat"):
            lax.fori_loop(0, _CS // 128, gather, 0)
        c = tid * _PLANES + (k >> 6)
        q = k & (_CPP - 1)
        pltpu.async_copy(obuf, out.at[sc, c, pl.ds(q * _CS, _CS)], semo)

    def o_drain(obuf, semo):
        pltpu.make_async_copy(obuf, out.at[sc, 0, pl.ds(0, _CS)],
                              semo).wait()

    pltpu.async_copy(m_src(0), mb0, semm0)
    pltpu.async_copy(m_src(1), mb1, semm1)

    def chunk_pair(kk, _):
        k0 = kk * 2
        with jax.named_scope("m_wait"):
            pltpu.make_async_copy(m_src(0), mb0, semm0).wait()

        @pl.when(kk > 0)
        def _():
            with jax.named_scope("odrain"):
                o_drain(ob0, semo0)

        emit(k0, mb0, ob0, semo0)

        @pl.when(k0 + 2 < _NCK)
        def _():
            pltpu.async_copy(m_src(k0 + 2), mb0, semm0)

        pltpu.make_async_copy(m_src(0), mb1, semm1).wait()

        @pl.when(kk > 0)
        def _():
            o_drain(ob1, semo1)

        emit(k0 + 1, mb1, ob1, semo1)

        @pl.when(k0 + 3 < _NCK)
        def _():
            pltpu.async_copy(m_src(k0 + 3), mb1, semm1)

        return 0

    lax.fori_loop(0, _NCK // 2, chunk_pair, 0)
    o_drain(ob0, semo0)
    o_drain(ob1, semo1)


@functools.cache
def _build_sc():
    @functools.partial(
        pl.kernel,
        out_type=(
            jax.ShapeDtypeStruct((_B, _C, _SPB), jnp.float32),
            jax.ShapeDtypeStruct((_S,), jnp.int32),
        ),
        mesh=plsc.VectorSubcoreMesh(core_axis_name="c", subcore_axis_name="s"),
        compiler_params=pltpu.CompilerParams(needs_layout_passes=False,
                                             use_tc_tiling_on_sc=False),
        scratch_types=[
            pltpu.VMEM((_HALF,), jnp.int32),
            pltpu.VMEM((_CH,), jnp.int32),
            pltpu.VMEM((_CH,), jnp.int32),
            pltpu.VMEM((_CS,), jnp.int32),
            pltpu.VMEM((_CS,), jnp.int32),
            pltpu.VMEM((_CS,), jnp.float32),
            pltpu.VMEM((_CS,), jnp.float32),
            pltpu.VMEM((_NPAD,), jnp.float32),
            pltpu.SemaphoreType.DMA,
            pltpu.SemaphoreType.DMA,
            pltpu.SemaphoreType.DMA,
            pltpu.SemaphoreType.DMA,
            pltpu.SemaphoreType.DMA,
            pltpu.SemaphoreType.DMA,
        ],
    )
    def _sparse_to_dense(dest, featT, out, m_out, *scratch):
        _sc_body(dest, featT, out, m_out, *scratch)

    return _sparse_to_dense


def kernel(features, batch_idx, coords):
    dest = ((batch_idx * _D + coords[:, 0]) * _D + coords[:, 1]) * _D \
        + coords[:, 2]
    featT = _transpose_features(features)
    dense, _ = _build_sc()(dest.astype(jnp.int32), featT)
    return dense.reshape(_B, _C, _D, _D, _D)


# speedup vs baseline: 13.4207x; 1.1589x over previous
import functools

import jax
import jax.numpy as jnp
from jax import lax
from jax.experimental import pallas as pl
from jax.experimental.pallas import tpu as pltpu
from jax.experimental.pallas import tpu_sc as plsc

_B = 2
_C = 64
_D = 64
_N = 100000
_S = _B * _D * _D * _D
_SPB = _D * _D * _D
_NT = 16
_SLOTS_T = _SPB // _NT
_NPAD = 100352
_SENT = _N
_CH = 2000
_NCHUNK = _N // _CH
_GRP = _CH // 16
_CS = 2048
_CPP = _SPB // _CS
_PLANES = 4
_NCK = _PLANES * _CPP
_TB = 1024


def _tc_transpose_body(x_ref, o_ref):
    i = pl.program_id(0)
    rows = jax.lax.broadcasted_iota(jnp.int32, (_TB, _C), 0) + i * _TB
    x = jnp.where(rows < _N, x_ref[...], 0.0)
    o_ref[...] = x.T


def _transpose_features(features):
    return pl.pallas_call(
        _tc_transpose_body,
        grid=(_NPAD // _TB,),
        in_specs=[pl.BlockSpec((_TB, _C), lambda i: (i, 0))],
        out_specs=pl.BlockSpec((_C, _TB), lambda i: (0, i)),
        out_shape=jax.ShapeDtypeStruct((_C, _NPAD), jnp.float32),
    )(features)


def _sc_body(dest, featT, out, m_out, mloc, db0, db1, mb0, mb1, ob0, ob1,
             row, semd0, semd1, semm0, semm1, semo0, semo1):
    sc = lax.axis_index("c")
    tid = lax.axis_index("s")
    mybase = sc * _SPB + tid * _SLOTS_T

    iota = lax.iota(jnp.int32, 16)
    sent = jnp.full((16,), _SENT, dtype=jnp.int32)

    def build_pass():
        lo = mybase

        def init(i, _):
            for u in range(8):
                mloc[pl.ds((i * 8 + u) * 16, 16)] = sent
            return 0

        lax.fori_loop(0, _SLOTS_T // 128, init, 0)

        def consume(t, dbuf):
            off = t * _CH

            def grp(i, _):
                ds = [dbuf[pl.ds((i * 5 + u) * 16, 16)] for u in range(5)]
                for u in range(5):
                    loc = ds[u] - lo
                    ok = (loc >= 0) & (loc < _SLOTS_T)
                    locc = loc & (_SLOTS_T - 1)
                    pidx = iota + (off + (i * 5 + u) * 16)
                    plsc.store_scatter(mloc, [locc], pidx, mask=ok)
                return 0

            lax.fori_loop(0, _GRP // 5, grp, 0)

        pltpu.async_copy(dest.at[pl.ds(0, _CH)], db0, semd0)
        pltpu.async_copy(dest.at[pl.ds(_CH, _CH)], db1, semd1)

        def pair(kk, _):
            t0 = kk * 2
            pltpu.make_async_copy(dest.at[pl.ds(0, _CH)], db0, semd0).wait()
            consume(t0, db0)

            @pl.when(t0 + 2 < _NCHUNK)
            def _():
                pltpu.async_copy(dest.at[pl.ds((t0 + 2) * _CH, _CH)], db0,
                                 semd0)

            pltpu.make_async_copy(dest.at[pl.ds(0, _CH)], db1, semd1).wait()
            consume(t0 + 1, db1)

            @pl.when(t0 + 3 < _NCHUNK)
            def _():
                pltpu.async_copy(dest.at[pl.ds((t0 + 3) * _CH, _CH)], db1,
                                 semd1)

            return 0

        lax.fori_loop(0, _NCHUNK // 2, pair, 0)
        pltpu.sync_copy(mloc, m_out.at[pl.ds(lo, _SLOTS_T)])

    with jax.named_scope("build"):
        build_pass()
        plsc.subcore_barrier()

    def m_src(k):
        q = k & (_CPP - 1)
        return m_out.at[pl.ds(sc * _SPB + q * _CS, _CS)]

    def plane_of(k):
        return lax.div(k, _CPP)

    def emit(k, mbuf, obuf, semo):
        @pl.when((k & (_CPP - 1)) == 0)
        def _():
            with jax.named_scope("row"):
                c = tid * _PLANES + plane_of(k)
                pltpu.sync_copy(featT.at[c], row)

        with jax.named_scope("gat"):
            @plsc.parallel_loop(0, _CS, step=16, unroll=8)
            def gather(g):
                idx = mbuf[pl.ds(g, 16)]
                obuf[pl.ds(g, 16)] = plsc.load_gather(row, [idx])

        c = tid * _PLANES + plane_of(k)
        q = k & (_CPP - 1)
        pltpu.async_copy(obuf, out.at[sc, c, pl.ds(q * _CS, _CS)], semo)

    def o_drain(obuf, semo):
        pltpu.make_async_copy(obuf, out.at[sc, 0, pl.ds(0, _CS)],
                              semo).wait()

    pltpu.async_copy(m_src(0), mb0, semm0)
    pltpu.async_copy(m_src(1), mb1, semm1)

    def chunk_pair(kk, _):
        k0 = kk * 2
        with jax.named_scope("m_wait"):
            pltpu.make_async_copy(m_src(0), mb0, semm0).wait()

        @pl.when(kk > 0)
        def _():
            with jax.named_scope("odrain"):
                o_drain(ob0, semo0)

        emit(k0, mb0, ob0, semo0)

        @pl.when(k0 + 2 < _NCK)
        def _():
            pltpu.async_copy(m_src(k0 + 2), mb0, semm0)

        pltpu.make_async_copy(m_src(0), mb1, semm1).wait()

        @pl.when(kk > 0)
        def _():
            o_drain(ob1, semo1)

        emit(k0 + 1, mb1, ob1, semo1)

        @pl.when(k0 + 3 < _NCK)
        def _():
            pltpu.async_copy(m_src(k0 + 3), mb1, semm1)

        return 0

    lax.fori_loop(0, _NCK // 2, chunk_pair, 0)
    o_drain(ob0, semo0)
    o_drain(ob1, semo1)


@functools.cache
def _build_sc():
    @functools.partial(
        pl.kernel,
        out_type=(
            jax.ShapeDtypeStruct((_B, _C, _SPB), jnp.float32),
            jax.ShapeDtypeStruct((_S,), jnp.int32),
        ),
        mesh=plsc.VectorSubcoreMesh(core_axis_name="c", subcore_axis_name="s"),
        compiler_params=pltpu.CompilerParams(needs_layout_passes=False,
                                             use_tc_tiling_on_sc=False),
        scratch_types=[
            pltpu.VMEM((_SLOTS_T,), jnp.int32),
            pltpu.VMEM((_CH,), jnp.int32),
            pltpu.VMEM((_CH,), jnp.int32),
            pltpu.VMEM((_CS,), jnp.int32),
            pltpu.VMEM((_CS,), jnp.int32),
            pltpu.VMEM((_CS,), jnp.float32),
            pltpu.VMEM((_CS,), jnp.float32),
            pltpu.VMEM((_NPAD,), jnp.float32),
            pltpu.SemaphoreType.DMA,
            pltpu.SemaphoreType.DMA,
            pltpu.SemaphoreType.DMA,
            pltpu.SemaphoreType.DMA,
            pltpu.SemaphoreType.DMA,
            pltpu.SemaphoreType.DMA,
        ],
    )
    def _sparse_to_dense(dest, featT, out, m_out, *scratch):
        _sc_body(dest, featT, out, m_out, *scratch)

    return _sparse_to_dense


def kernel(features, batch_idx, coords):
    dest = ((batch_idx * _D + coords[:, 0]) * _D + coords[:, 1]) * _D \
        + coords[:, 2]
    featT = _transpose_features(features)
    dense, _ = _build_sc()(dest.astype(jnp.int32), featT)
    return dense.reshape(_B, _C, _D, _D, _D)


# speedup vs baseline: 14.5601x; 1.0849x over previous
import functools

import jax
import jax.numpy as jnp
from jax import lax
from jax.experimental import pallas as pl
from jax.experimental.pallas import tpu as pltpu
from jax.experimental.pallas import tpu_sc as plsc

_B = 2
_C = 64
_D = 64
_N = 100000
_S = _B * _D * _D * _D
_SPB = _D * _D * _D
_NT = 16
_SLOTS_T = _SPB // _NT
_NPAD = 100352
_SENT = _N
_CH = 400
_NCHUNK = _N // _CH
_GRP = _CH // 16
_CS = 2048
_CPP = _SPB // _CS
_PLANES = 4
_NCK = _PLANES * _CPP
_TB = 1024


def _tc_transpose_body(x_ref, o_ref):
    i = pl.program_id(0)
    rows = jax.lax.broadcasted_iota(jnp.int32, (_TB, _C), 0) + i * _TB
    x = jnp.where(rows < _N, x_ref[...], 0.0)
    o_ref[...] = x.T


def _transpose_features(features):
    return pl.pallas_call(
        _tc_transpose_body,
        grid=(_NPAD // _TB,),
        in_specs=[pl.BlockSpec((_TB, _C), lambda i: (i, 0))],
        out_specs=pl.BlockSpec((_C, _TB), lambda i: (0, i)),
        out_shape=jax.ShapeDtypeStruct((_C, _NPAD), jnp.float32),
    )(features)


def _sc_body(dest, featT, out, m_out, mloc, db0, db1, mb0, mb1, mb2, mb3,
             ob0, ob1, row, semd0, semd1, semm0, semm1, semm2, semm3,
             semo0, semo1):
    sc = lax.axis_index("c")
    tid = lax.axis_index("s")
    mybase = sc * _SPB + tid * _SLOTS_T

    iota = lax.iota(jnp.int32, 16)
    sent = jnp.full((16,), _SENT, dtype=jnp.int32)

    def build_pass():
        lo = mybase

        def init(i, _):
            for u in range(8):
                mloc[pl.ds((i * 8 + u) * 16, 16)] = sent
            return 0

        lax.fori_loop(0, _SLOTS_T // 128, init, 0)

        def consume(t, dbuf):
            off = t * _CH

            def grp(i, _):
                ds = [dbuf[pl.ds((i * 5 + u) * 16, 16)] for u in range(5)]
                for u in range(5):
                    loc = ds[u] - lo
                    ok = (loc >= 0) & (loc < _SLOTS_T)
                    locc = loc & (_SLOTS_T - 1)
                    pidx = iota + (off + (i * 5 + u) * 16)
                    plsc.store_scatter(mloc, [locc], pidx, mask=ok)
                return 0

            lax.fori_loop(0, _GRP // 5, grp, 0)

        pltpu.async_copy(dest.at[pl.ds(0, _CH)], db0, semd0)
        pltpu.async_copy(dest.at[pl.ds(_CH, _CH)], db1, semd1)

        def pair(kk, _):
            t0 = kk * 2
            pltpu.make_async_copy(dest.at[pl.ds(0, _CH)], db0, semd0).wait()
            consume(t0, db0)

            @pl.when(t0 + 2 < _NCHUNK)
            def _():
                pltpu.async_copy(dest.at[pl.ds((t0 + 2) * _CH, _CH)], db0,
                                 semd0)

            pltpu.make_async_copy(dest.at[pl.ds(0, _CH)], db1, semd1).wait()
            consume(t0 + 1, db1)

            @pl.when(t0 + 3 < _NCHUNK)
            def _():
                pltpu.async_copy(dest.at[pl.ds((t0 + 3) * _CH, _CH)], db1,
                                 semd1)

            return 0

        lax.fori_loop(0, _NCHUNK // 2, pair, 0)
        pltpu.sync_copy(mloc, m_out.at[pl.ds(lo, _SLOTS_T)])

    with jax.named_scope("build"):
        build_pass()
        plsc.subcore_barrier()

    mbs = (mb0, mb1, mb2, mb3)
    semms = (semm0, semm1, semm2, semm3)
    obs = (ob0, ob1)
    semos = (semo0, semo1)

    def m_src(k):
        q = k & (_CPP - 1)
        return m_out.at[pl.ds(sc * _SPB + q * _CS, _CS)]

    def plane_of(k):
        return lax.div(k, _CPP)

    def emit(k, mbuf, obuf, semo):
        @pl.when((k & (_CPP - 1)) == 0)
        def _():
            with jax.named_scope("row"):
                c = tid * _PLANES + plane_of(k)
                pltpu.sync_copy(featT.at[c], row)

        with jax.named_scope("gat"):
            @plsc.parallel_loop(0, 32, step=1, unroll=8)
            def gather(r):
                for u4 in range(4):
                    idx = mbuf[pl.ds(r * 64 + u4 * 16, 16)]
                    obuf[r, pl.ds(u4 * 16, 16)] = plsc.load_gather(row,
                                                                   [idx])

        c = tid * _PLANES + plane_of(k)
        q = k & (_CPP - 1)
        pltpu.async_copy(
            obuf, out.at[sc, c, q >> 1, pl.ds((q & 1) * 32, 32)], semo)

    def o_drain(obuf, semo):
        pltpu.make_async_copy(obuf, out.at[sc, 0, 0, pl.ds(0, 32)],
                              semo).wait()

    for u in range(4):
        pltpu.async_copy(m_src(u), mbs[u], semms[u])

    def chunk_quad(kk, _):
        k0 = kk * 4
        for u in range(4):
            k = k0 + u
            with jax.named_scope("m_wait"):
                pltpu.make_async_copy(m_src(0), mbs[u], semms[u]).wait()

            if u < 2:
                @pl.when(kk > 0)
                def _():
                    with jax.named_scope("odrain"):
                        o_drain(obs[u & 1], semos[u & 1])
            else:
                with jax.named_scope("odrain"):
                    o_drain(obs[u & 1], semos[u & 1])

            emit(k, mbs[u], obs[u & 1], semos[u & 1])

            @pl.when(k + 4 < _NCK)
            def _():
                pltpu.async_copy(m_src(k + 4), mbs[u], semms[u])

        return 0

    lax.fori_loop(0, _NCK // 4, chunk_quad, 0)
    o_drain(ob0, semo0)
    o_drain(ob1, semo1)


@functools.cache
def _build_sc():
    @functools.partial(
        pl.kernel,
        out_type=(
            jax.ShapeDtypeStruct((_B, _C, _D, _D, _D), jnp.float32),
            jax.ShapeDtypeStruct((_S,), jnp.int32),
        ),
        mesh=plsc.VectorSubcoreMesh(core_axis_name="c", subcore_axis_name="s"),
        compiler_params=pltpu.CompilerParams(needs_layout_passes=False,
                                             use_tc_tiling_on_sc=False),
        scratch_types=[
            pltpu.VMEM((_SLOTS_T,), jnp.int32),
            pltpu.VMEM((_CH,), jnp.int32),
            pltpu.VMEM((_CH,), jnp.int32),
            pltpu.VMEM((_CS,), jnp.int32),
            pltpu.VMEM((_CS,), jnp.int32),
            pltpu.VMEM((_CS,), jnp.int32),
            pltpu.VMEM((_CS,), jnp.int32),
            pltpu.VMEM((32, 64), jnp.float32),
            pltpu.VMEM((32, 64), jnp.float32),
            pltpu.VMEM((_NPAD,), jnp.float32),
            pltpu.SemaphoreType.DMA,
            pltpu.SemaphoreType.DMA,
            pltpu.SemaphoreType.DMA,
            pltpu.SemaphoreType.DMA,
            pltpu.SemaphoreType.DMA,
            pltpu.SemaphoreType.DMA,
            pltpu.SemaphoreType.DMA,
            pltpu.SemaphoreType.DMA,
        ],
    )
    def _sparse_to_dense(dest, featT, out, m_out, *scratch):
        _sc_body(dest, featT, out, m_out, *scratch)

    return _sparse_to_dense


def kernel(features, batch_idx, coords):
    dest = ((batch_idx * _D + coords[:, 0]) * _D + coords[:, 1]) * _D \
        + coords[:, 2]
    featT = _transpose_features(features)
    dense, _ = _build_sc()(dest.astype(jnp.int32), featT)
    return dense


# speedup vs baseline: 15.7833x; 1.0840x over previous
import functools

import jax
import jax.numpy as jnp
from jax import lax
from jax.experimental import pallas as pl
from jax.experimental.pallas import tpu as pltpu
from jax.experimental.pallas import tpu_sc as plsc

_B = 2
_C = 64
_D = 64
_N = 100000
_S = _B * _D * _D * _D
_SPB = _D * _D * _D
_NT = 16
_SLOTS_T = _SPB // _NT
_NPAD = 100352
_SENT = _N
_NP2 = 102400
_CH = 2048
_NCHUNK = _NP2 // _CH
_GRP = _CH // 16
_CS = 2048
_CPP = _SPB // _CS
_PLANES = 4
_NCK = _PLANES * _CPP
_TB = 1024


def _tc_transpose_body(x_ref, o_ref):
    i = pl.program_id(0)
    rows = jax.lax.broadcasted_iota(jnp.int32, (_TB, _C), 0) + i * _TB
    x = jnp.where(rows < _N, x_ref[...], 0.0)
    o_ref[...] = x.T


def _transpose_features(features):
    return pl.pallas_call(
        _tc_transpose_body,
        grid=(_NPAD // _TB,),
        in_specs=[pl.BlockSpec((_TB, _C), lambda i: (i, 0))],
        out_specs=pl.BlockSpec((_C, _TB), lambda i: (0, i)),
        out_shape=jax.ShapeDtypeStruct((_C, _NPAD), jnp.float32),
    )(features)


def _sc_body(dest, featT, out, m_out, mloc, mb0, mb1, mb2, mb3,
             ob0, ob1, row, semm0, semm1, semm2, semm3, semo0, semo1):
    sc = lax.axis_index("c")
    tid = lax.axis_index("s")
    mybase = sc * _SPB + tid * _SLOTS_T

    iota = lax.iota(jnp.int32, 16)
    sent = jnp.full((16,), _SENT, dtype=jnp.int32)

    def build_pass():
        lo = mybase

        def init(i, _):
            for u in range(8):
                mloc[pl.ds((i * 8 + u) * 16, 16)] = sent
            return 0

        lax.fori_loop(0, _SLOTS_T // 128, init, 0)

        def consume(t, dbuf):
            off = t * _CH

            def grp(i, _):
                ds = [dbuf[pl.ds((i * 4 + u) * 16, 16)] for u in range(4)]
                for u in range(4):
                    loc = ds[u] - lo
                    ok = (loc >= 0) & (loc < _SLOTS_T)
                    locc = loc & (_SLOTS_T - 1)
                    pidx = iota + (off + (i * 4 + u) * 16)
                    plsc.store_scatter(mloc, [locc], pidx, mask=ok)
                return 0

            lax.fori_loop(0, _GRP // 4, grp, 0)

        pltpu.async_copy(dest.at[pl.ds(0, _CH)], mb0, semm0)
        pltpu.async_copy(dest.at[pl.ds(_CH, _CH)], mb1, semm1)

        def pair(kk, _):
            t0 = kk * 2
            pltpu.make_async_copy(dest.at[pl.ds(0, _CH)], mb0, semm0).wait()
            consume(t0, mb0)

            @pl.when(t0 + 2 < _NCHUNK)
            def _():
                pltpu.async_copy(dest.at[pl.ds((t0 + 2) * _CH, _CH)], mb0,
                                 semm0)

            pltpu.make_async_copy(dest.at[pl.ds(0, _CH)], mb1, semm1).wait()
            consume(t0 + 1, mb1)

            @pl.when(t0 + 3 < _NCHUNK)
            def _():
                pltpu.async_copy(dest.at[pl.ds((t0 + 3) * _CH, _CH)], mb1,
                                 semm1)

            return 0

        lax.fori_loop(0, _NCHUNK // 2, pair, 0)
        pltpu.sync_copy(mloc, m_out.at[pl.ds(lo, _SLOTS_T)])

    with jax.named_scope("build"):
        build_pass()
        plsc.subcore_barrier()

    mbs = (mb0, mb1, mb2, mb3)
    semms = (semm0, semm1, semm2, semm3)
    obs = (ob0, ob1)
    semos = (semo0, semo1)

    def m_src(k):
        q = k & (_CPP - 1)
        return m_out.at[pl.ds(sc * _SPB + q * _CS, _CS)]

    def plane_of(k):
        return lax.div(k, _CPP)

    def emit(k, mbuf, obuf, semo):
        @pl.when((k & (_CPP - 1)) == 0)
        def _():
            with jax.named_scope("row"):
                c = tid * _PLANES + plane_of(k)
                pltpu.sync_copy(featT.at[c], row)

        with jax.named_scope("gat"):
            @plsc.parallel_loop(0, 32, step=1, unroll=8)
            def gather(r):
                for u4 in range(4):
                    idx = mbuf[pl.ds(r * 64 + u4 * 16, 16)]
                    obuf[r, pl.ds(u4 * 16, 16)] = plsc.load_gather(row,
                                                                   [idx])

        c = tid * _PLANES + plane_of(k)
        q = k & (_CPP - 1)
        pltpu.async_copy(
            obuf, out.at[sc, c, q >> 1, pl.ds((q & 1) * 32, 32)], semo)

    def o_drain(obuf, semo):
        pltpu.make_async_copy(obuf, out.at[sc, 0, 0, pl.ds(0, 32)],
                              semo).wait()

    for u in range(4):
        pltpu.async_copy(m_src(u), mbs[u], semms[u])

    def chunk_quad(kk, _):
        k0 = kk * 4
        for u in range(4):
            k = k0 + u
            with jax.named_scope("m_wait"):
                pltpu.make_async_copy(m_src(0), mbs[u], semms[u]).wait()

            if u < 2:
                @pl.when(kk > 0)
                def _():
                    with jax.named_scope("odrain"):
                        o_drain(obs[u & 1], semos[u & 1])
            else:
                with jax.named_scope("odrain"):
                    o_drain(obs[u & 1], semos[u & 1])

            emit(k, mbs[u], obs[u & 1], semos[u & 1])

            @pl.when(k + 4 < _NCK)
            def _():
                pltpu.async_copy(m_src(k + 4), mbs[u], semms[u])

        return 0

    lax.fori_loop(0, _NCK // 4, chunk_quad, 0)
    o_drain(ob0, semo0)
    o_drain(ob1, semo1)


@functools.cache
def _build_sc():
    @functools.partial(
        pl.kernel,
        out_type=(
            jax.ShapeDtypeStruct((_B, _C, _D, _D, _D), jnp.float32),
            jax.ShapeDtypeStruct((_S,), jnp.int32),
        ),
        mesh=plsc.VectorSubcoreMesh(core_axis_name="c", subcore_axis_name="s"),
        compiler_params=pltpu.CompilerParams(needs_layout_passes=False,
                                             use_tc_tiling_on_sc=False),
        scratch_types=[
            pltpu.VMEM((_SLOTS_T,), jnp.int32),
            pltpu.VMEM((_CS,), jnp.int32),
            pltpu.VMEM((_CS,), jnp.int32),
            pltpu.VMEM((_CS,), jnp.int32),
            pltpu.VMEM((_CS,), jnp.int32),
            pltpu.VMEM((32, 64), jnp.float32),
            pltpu.VMEM((32, 64), jnp.float32),
            pltpu.VMEM((_NPAD,), jnp.float32),
            pltpu.SemaphoreType.DMA,
            pltpu.SemaphoreType.DMA,
            pltpu.SemaphoreType.DMA,
            pltpu.SemaphoreType.DMA,
            pltpu.SemaphoreType.DMA,
            pltpu.SemaphoreType.DMA,
        ],
    )
    def _sparse_to_dense(dest, featT, out, m_out, *scratch):
        _sc_body(dest, featT, out, m_out, *scratch)

    return _sparse_to_dense


def kernel(features, batch_idx, coords):
    dest = ((batch_idx * _D + coords[:, 0]) * _D + coords[:, 1]) * _D \
        + coords[:, 2]
    dest = jnp.concatenate(
        [dest, jnp.full((_NP2 - _N,), _S, dtype=jnp.int32)])
    featT = _transpose_features(features)
    dense, _ = _build_sc()(dest.astype(jnp.int32), featT)
    return dense


# speedup vs baseline: 15.8018x; 1.0012x over previous
import functools

import jax
import jax.numpy as jnp
from jax import lax
from jax.experimental import pallas as pl
from jax.experimental.pallas import tpu as pltpu
from jax.experimental.pallas import tpu_sc as plsc

_B = 2
_C = 64
_D = 64
_N = 100000
_S = _B * _D * _D * _D
_SPB = _D * _D * _D
_NT = 16
_SLOTS_T = _SPB // _NT
_NPAD = 100352
_SENT = _N
_NP2 = 102400
_CH = 2048
_NCHUNK = _NP2 // _CH
_GRP = _CH // 16
_CS = 2048
_CPP = _SPB // _CS
_PLANES = 4
_NCK = _PLANES * _CPP
_TB = 1024


def _tc_transpose_body(x_ref, o_ref):
    i = pl.program_id(0)
    rows = jax.lax.broadcasted_iota(jnp.int32, (_TB, _C), 0) + i * _TB
    x = jnp.where(rows < _N, x_ref[...], 0.0)
    o_ref[...] = x.T


def _transpose_features(features):
    return pl.pallas_call(
        _tc_transpose_body,
        grid=(_NPAD // _TB,),
        in_specs=[pl.BlockSpec((_TB, _C), lambda i: (i, 0))],
        out_specs=pl.BlockSpec((_C, _TB), lambda i: (0, i)),
        out_shape=jax.ShapeDtypeStruct((_C, _NPAD), jnp.float32),
    )(features)


def _sc_body(dest, featT, out, m_out, mloc, mb0, mb1, mb2, mb3,
             ob0, ob1, row, semm0, semm1, semm2, semm3, semo0, semo1):
    sc = lax.axis_index("c")
    tid = lax.axis_index("s")
    mybase = sc * _SPB + tid * _SLOTS_T

    iota = lax.iota(jnp.int32, 16)
    sent = jnp.full((16,), _SENT, dtype=jnp.int32)

    def build_pass():
        lo = mybase

        def init(i, _):
            for u in range(8):
                mloc[pl.ds((i * 8 + u) * 16, 16)] = sent
            return 0

        lax.fori_loop(0, _SLOTS_T // 128, init, 0)

        def consume(t, dbuf):
            off = t * _CH

            def grp(i, _):
                ds = [dbuf[pl.ds((i * 4 + u) * 16, 16)] for u in range(4)]
                for u in range(4):
                    loc = ds[u] - lo
                    ok = (loc >= 0) & (loc < _SLOTS_T)
                    locc = loc & (_SLOTS_T - 1)
                    pidx = iota + (off + (i * 4 + u) * 16)
                    plsc.store_scatter(mloc, [locc], pidx, mask=ok)
                return 0

            lax.fori_loop(0, _GRP // 4, grp, 0)

        pltpu.async_copy(dest.at[pl.ds(0, _CH)], mb0, semm0)
        pltpu.async_copy(dest.at[pl.ds(_CH, _CH)], mb1, semm1)

        def pair(kk, _):
            t0 = kk * 2
            pltpu.make_async_copy(dest.at[pl.ds(0, _CH)], mb0, semm0).wait()
            consume(t0, mb0)

            @pl.when(t0 + 2 < _NCHUNK)
            def _():
                pltpu.async_copy(dest.at[pl.ds((t0 + 2) * _CH, _CH)], mb0,
                                 semm0)

            pltpu.make_async_copy(dest.at[pl.ds(0, _CH)], mb1, semm1).wait()
            consume(t0 + 1, mb1)

            @pl.when(t0 + 3 < _NCHUNK)
            def _():
                pltpu.async_copy(dest.at[pl.ds((t0 + 3) * _CH, _CH)], mb1,
                                 semm1)

            return 0

        lax.fori_loop(0, _NCHUNK // 2, pair, 0)
        pltpu.sync_copy(mloc, m_out.at[pl.ds(lo, _SLOTS_T)])

    with jax.named_scope("build"):
        build_pass()
        plsc.subcore_barrier()

    mbs = (mb0, mb1, mb2, mb3)
    semms = (semm0, semm1, semm2, semm3)
    obs = (ob0, ob1)
    semos = (semo0, semo1)

    def m_src(k):
        q = k & (_CPP - 1)
        return m_out.at[pl.ds(sc * _SPB + q * _CS, _CS)]

    def plane_of(k):
        return lax.div(k, _CPP)

    def emit(k, mbuf, obuf, semo):
        @pl.when((k & (_CPP - 1)) == 0)
        def _():
            with jax.named_scope("row"):
                c = tid * _PLANES + plane_of(k)
                pltpu.sync_copy(featT.at[pl.ds(c * _NPAD, _NPAD)], row)

        with jax.named_scope("gat"):
            @plsc.parallel_loop(0, 32, step=1, unroll=8)
            def gather(r):
                for u4 in range(4):
                    idx = mbuf[pl.ds(r * 64 + u4 * 16, 16)]
                    obuf[r, pl.ds(u4 * 16, 16)] = plsc.load_gather(row,
                                                                   [idx])

        c = tid * _PLANES + plane_of(k)
        q = k & (_CPP - 1)
        pltpu.async_copy(
            obuf, out.at[sc, c, q >> 1, pl.ds((q & 1) * 32, 32)], semo)

    def o_drain(obuf, semo):
        pltpu.make_async_copy(obuf, out.at[sc, 0, 0, pl.ds(0, 32)],
                              semo).wait()

    for u in range(4):
        pltpu.async_copy(m_src(u), mbs[u], semms[u])

    def chunk_quad(kk, _):
        k0 = kk * 4
        for u in range(4):
            k = k0 + u
            with jax.named_scope("m_wait"):
                pltpu.make_async_copy(m_src(0), mbs[u], semms[u]).wait()

            if u < 2:
                @pl.when(kk > 0)
                def _():
                    with jax.named_scope("odrain"):
                        o_drain(obs[u & 1], semos[u & 1])
            else:
                with jax.named_scope("odrain"):
                    o_drain(obs[u & 1], semos[u & 1])

            emit(k, mbs[u], obs[u & 1], semos[u & 1])

            @pl.when(k + 4 < _NCK)
            def _():
                pltpu.async_copy(m_src(k + 4), mbs[u], semms[u])

        return 0

    lax.fori_loop(0, _NCK // 4, chunk_quad, 0)
    o_drain(ob0, semo0)
    o_drain(ob1, semo1)


@functools.cache
def _build_sc():
    @functools.partial(
        pl.kernel,
        out_type=(
            jax.ShapeDtypeStruct((_B, _C, _D, _D, _D), jnp.float32),
            jax.ShapeDtypeStruct((_S,), jnp.int32),
        ),
        mesh=plsc.VectorSubcoreMesh(core_axis_name="c", subcore_axis_name="s"),
        compiler_params=pltpu.CompilerParams(needs_layout_passes=False,
                                             use_tc_tiling_on_sc=False),
        scratch_types=[
            pltpu.VMEM((_SLOTS_T,), jnp.int32),
            pltpu.VMEM((_CS,), jnp.int32),
            pltpu.VMEM((_CS,), jnp.int32),
            pltpu.VMEM((_CS,), jnp.int32),
            pltpu.VMEM((_CS,), jnp.int32),
            pltpu.VMEM((32, 64), jnp.float32),
            pltpu.VMEM((32, 64), jnp.float32),
            pltpu.VMEM((_NPAD,), jnp.float32),
            pltpu.SemaphoreType.DMA,
            pltpu.SemaphoreType.DMA,
            pltpu.SemaphoreType.DMA,
            pltpu.SemaphoreType.DMA,
            pltpu.SemaphoreType.DMA,
            pltpu.SemaphoreType.DMA,
        ],
    )
    def _sparse_to_dense(dest, featT, out, m_out, *scratch):
        _sc_body(dest, featT, out, m_out, *scratch)

    return _sparse_to_dense


def kernel(features, batch_idx, coords):
    dest = ((batch_idx * _D + coords[:, 0]) * _D + coords[:, 1]) * _D \
        + coords[:, 2]
    dest = jnp.concatenate(
        [dest, jnp.full((_NP2 - _N,), _S, dtype=jnp.int32)])
    featT = _transpose_features(features).reshape(-1)
    dense, _ = _build_sc()(dest.astype(jnp.int32), featT)
    return dense
